# baseline (speedup 1.0000x reference)
import jax
import jax.numpy as jnp
from jax.experimental import pallas as pl

DEPTH = 1000
BATCH = 1024
FEATS = 26
BLOCK_K = 200


def _onehot_t_block(idx_ref, out_ref):
    j = pl.program_id(1)
    idx = idx_ref[...]
    k = jax.lax.broadcasted_iota(jnp.int32, (1, BLOCK_K, BATCH), 1) + j * BLOCK_K
    out_ref[...] = (k == idx).astype(jnp.float32)


def kernel(indices):
    idx_t = indices.T.reshape(FEATS, 1, BATCH)
    out_t = pl.pallas_call(
        _onehot_t_block,
        grid=(FEATS, DEPTH // BLOCK_K),
        in_specs=[pl.BlockSpec((1, 1, BATCH), lambda i, j: (i, 0, 0))],
        out_specs=pl.BlockSpec((1, BLOCK_K, BATCH), lambda i, j: (i, j, 0)),
        out_shape=jax.ShapeDtypeStruct((FEATS, DEPTH, BATCH), jnp.float32),
    )(idx_t)
    return jnp.transpose(out_t, (2, 0, 1))


# speedup vs baseline: 1.9785x; 1.9785x over previous
import jax
import jax.numpy as jnp
from jax.experimental import pallas as pl
from jax.experimental.pallas import tpu as pltpu

DEPTH = 1000
BATCH = 1024
FEATS = 26
NBUF = 4


def _body(idx_ref, out_ref, scratch, sems):
    i = pl.program_id(0)
    slot = jax.lax.rem(i, NBUF)

    @pl.when(i >= NBUF)
    def _wait_prev():
        pltpu.make_async_copy(
            scratch.at[slot], out_ref.at[i - NBUF], sems.at[slot]
        ).wait()

    idx = idx_ref[...]
    k = jax.lax.broadcasted_iota(jnp.int32, (DEPTH, BATCH), 0)
    scratch[slot] = (k == idx[0]).astype(jnp.float32)

    pltpu.make_async_copy(scratch.at[slot], out_ref.at[i], sems.at[slot]).start()

    @pl.when(i == FEATS - 1)
    def _drain():
        for d in range(NBUF):
            j = FEATS - NBUF + d
            pltpu.make_async_copy(
                scratch.at[j % NBUF], out_ref.at[j], sems.at[j % NBUF]
            ).wait()


def kernel(indices):
    idx_t = indices.T.reshape(FEATS, 1, BATCH)
    out_t = pl.pallas_call(
        _body,
        grid=(FEATS,),
        in_specs=[pl.BlockSpec((1, 1, BATCH), lambda i: (i, 0, 0))],
        out_specs=pl.BlockSpec(memory_space=pl.ANY),
        out_shape=jax.ShapeDtypeStruct((FEATS, DEPTH, BATCH), jnp.float32),
        scratch_shapes=[
            pltpu.VMEM((NBUF, DEPTH, BATCH), jnp.float32),
            pltpu.SemaphoreType.DMA((NBUF,)),
        ],
    )(idx_t)
    return jnp.transpose(out_t, (2, 0, 1))
